# baseline (speedup 1.0000x reference)
import functools

import jax
import jax.numpy as jnp
from jax import lax
from jax.experimental import pallas as pl
from jax.experimental.pallas import tpu as pltpu
from jax.experimental.pallas import tpu_sc as plsc

NC = 2
NS = 16
NW = NC * NS
CHUNK = 128
PHASES = 4


def _make_sc_segment_sum(n_pad, d, pc):
    mesh = plsc.VectorSubcoreMesh(core_axis_name="c", subcore_axis_name="s")
    rows_per_tile = n_pad // NS

    @functools.partial(
        pl.kernel,
        out_type=jax.ShapeDtypeStruct((NC, n_pad, d), jnp.float32),
        mesh=mesh,
        scratch_types=[
            pltpu.VMEM((pc, CHUNK), jnp.int32),
            pltpu.VMEM((pc, CHUNK), jnp.int32),
            pltpu.VMEM((CHUNK, d), jnp.float32),
            pltpu.VMEM((CHUNK, d), jnp.float32),
            pltpu.VMEM_SHARED((n_pad, d), jnp.float32),
            pltpu.SemaphoreType.DMA,
            pltpu.SemaphoreType.DMA,
        ],
    )
    def k(h_hbm, zeros_hbm, src_hbm, dst_hbm, out_hbm,
          src_v, dst_v, buf_a, buf_b, acc, sem_a, sem_b):
        cid = lax.axis_index("c")
        sid = lax.axis_index("s")
        wid = cid * NS + sid
        base = sid * rows_per_tile
        @pl.when(cid == 0)
        def _():
            pltpu.sync_copy(h_hbm.at[pl.ds(base, rows_per_tile)],
                            acc.at[pl.ds(base, rows_per_tile)])

        @pl.when(cid != 0)
        def _():
            pltpu.sync_copy(zeros_hbm.at[pl.ds(base, rows_per_tile)],
                            acc.at[pl.ds(base, rows_per_tile)])

        plsc.subcore_barrier()

        def g_start(j, buf, sem):
            pltpu.async_copy(h_hbm.at[src_v.at[j]], buf, sem)

        def g_wait(j, buf, sem):
            pltpu.make_async_copy(h_hbm.at[src_v.at[j]], buf, sem).wait()

        def scat(j, buf):
            pltpu.sync_copy(buf, acc.at[dst_v.at[j]], add=True)

        for ph in range(PHASES):
            pltpu.sync_copy(src_hbm.at[wid].at[ph], src_v)
            pltpu.sync_copy(dst_hbm.at[wid].at[ph], dst_v)

            g_start(0, buf_a, sem_a)

            def body(jj, carry):
                j0 = 2 * jj
                g_start(j0 + 1, buf_b, sem_b)
                g_wait(j0, buf_a, sem_a)
                scat(j0, buf_a)

                @pl.when(jj < pc // 2 - 1)
                def _():
                    g_start(j0 + 2, buf_a, sem_a)

                g_wait(j0 + 1, buf_b, sem_b)
                scat(j0 + 1, buf_b)
                return carry

            lax.fori_loop(0, pc // 2, body, 0)
        plsc.subcore_barrier()
        pltpu.sync_copy(acc.at[pl.ds(base, rows_per_tile)],
                        out_hbm.at[cid].at[pl.ds(base, rows_per_tile)])

    return k


def _make_mlp(n_pad, d, bm, n_out, final):
    inv_std = float((1.0 + 1e-5) ** -0.5)

    def body(p_ref, w1_ref, b1_ref, w2_ref, b2_ref, g_ref, be_ref, o_ref):
        z = p_ref[0] + p_ref[1]
        y = jnp.dot(z, w1_ref[...], preferred_element_type=jnp.float32)
        y = jnp.maximum(y + b1_ref[...], 0.0)
        y = jnp.dot(y, w2_ref[...], preferred_element_type=jnp.float32)
        y = y + b2_ref[...]
        y = y * (g_ref[...] * inv_std) + be_ref[...]
        if not final:
            y = jnp.maximum(y, 0.0)
        o_ref[...] = y

    grid = -(-n_out // bm)
    full = lambda i: (0, 0)
    return pl.pallas_call(
        body,
        grid=(grid,),
        in_specs=[
            pl.BlockSpec((NC, bm, d), lambda i: (0, i, 0)),
            pl.BlockSpec((d, d), full),
            pl.BlockSpec((d,), lambda i: (0,)),
            pl.BlockSpec((d, d), full),
            pl.BlockSpec((d,), lambda i: (0,)),
            pl.BlockSpec((d,), lambda i: (0,)),
            pl.BlockSpec((d,), lambda i: (0,)),
        ],
        out_specs=pl.BlockSpec((bm, d), lambda i: (i, 0)),
        out_shape=jax.ShapeDtypeStruct((n_out, d), jnp.float32),
    )


def kernel(x, edge_index, params):
    n, d = x.shape
    e = edge_index.shape[1]
    n_pad = ((n + 1 + 127) // 128) * 128
    chunks = -(-e // (NW * CHUNK))
    pc = -(-chunks // PHASES)
    pc = ((pc + 1) // 2) * 2
    chunks = PHASES * pc
    e_pad = NW * chunks * CHUNK

    bm = n_pad
    while bm > 4096 or bm % 8:
        bm //= 2

    pad_idx = n + (jnp.arange(e_pad - e, dtype=jnp.int32) % (n_pad - n))
    src = jnp.concatenate([edge_index[0], pad_idx]).reshape(NW, PHASES, pc, CHUNK)
    dst = jnp.concatenate([edge_index[1], pad_idx]).reshape(NW, PHASES, pc, CHUNK)

    h = jnp.concatenate([x, jnp.zeros((n_pad - n, d), jnp.float32)])
    zeros = jnp.zeros((n_pad, d), jnp.float32)

    seg = _make_sc_segment_sum(n_pad, d, pc)
    for i, (w1, b1, w2, b2, gamma, beta) in enumerate(params):
        final = i == len(params) - 1
        p = seg(h, zeros, src, dst)
        mlp = _make_mlp(n_pad, d, bm, n if final else n_pad, final)
        h = mlp(p, w1, b1, w2, b2, gamma, beta)
    return h


# speedup vs baseline: 1.0676x; 1.0676x over previous
import functools

import jax
import jax.numpy as jnp
from jax import lax
from jax.experimental import pallas as pl
from jax.experimental.pallas import tpu as pltpu
from jax.experimental.pallas import tpu_sc as plsc

NC = 2
NS = 16
NW = NC * NS
CHUNK = 128
PHASES = 2


def _make_sc_segment_sum(n_pad, d, pc):
    mesh = plsc.VectorSubcoreMesh(core_axis_name="c", subcore_axis_name="s")
    rows_per_tile = n_pad // NS

    @functools.partial(
        pl.kernel,
        out_type=jax.ShapeDtypeStruct((NC, n_pad, d), jnp.float32),
        mesh=mesh,
        scratch_types=[
            pltpu.VMEM((pc, CHUNK), jnp.int32),
            pltpu.VMEM((pc, CHUNK), jnp.int32),
            pltpu.VMEM((CHUNK, d), jnp.float32),
            pltpu.VMEM((CHUNK, d), jnp.float32),
            pltpu.VMEM_SHARED((n_pad, d), jnp.float32),
            pltpu.SemaphoreType.DMA,
            pltpu.SemaphoreType.DMA,
        ],
    )
    def k(h_hbm, zeros_hbm, src_hbm, dst_hbm, out_hbm,
          src_v, dst_v, buf_a, buf_b, acc, sem_a, sem_b):
        cid = lax.axis_index("c")
        sid = lax.axis_index("s")
        wid = cid * NS + sid
        base = sid * rows_per_tile
        @pl.when(cid == 0)
        def _():
            pltpu.sync_copy(h_hbm.at[pl.ds(base, rows_per_tile)],
                            acc.at[pl.ds(base, rows_per_tile)])

        @pl.when(cid != 0)
        def _():
            pltpu.sync_copy(zeros_hbm.at[pl.ds(base, rows_per_tile)],
                            acc.at[pl.ds(base, rows_per_tile)])

        plsc.subcore_barrier()

        def g_start(j, buf, sem):
            pltpu.async_copy(h_hbm.at[src_v.at[j]], buf, sem)

        def g_wait(j, buf, sem):
            pltpu.make_async_copy(h_hbm.at[src_v.at[j]], buf, sem).wait()

        def scat(j, buf):
            pltpu.sync_copy(buf, acc.at[dst_v.at[j]], add=True)

        for ph in range(PHASES):
            pltpu.sync_copy(src_hbm.at[wid].at[ph], src_v)
            pltpu.sync_copy(dst_hbm.at[wid].at[ph], dst_v)

            g_start(0, buf_a, sem_a)

            def body(jj, carry):
                j0 = 2 * jj
                g_start(j0 + 1, buf_b, sem_b)
                g_wait(j0, buf_a, sem_a)
                scat(j0, buf_a)

                @pl.when(jj < pc // 2 - 1)
                def _():
                    g_start(j0 + 2, buf_a, sem_a)

                g_wait(j0 + 1, buf_b, sem_b)
                scat(j0 + 1, buf_b)
                return carry

            lax.fori_loop(0, pc // 2, body, 0)
        plsc.subcore_barrier()
        pltpu.sync_copy(acc.at[pl.ds(base, rows_per_tile)],
                        out_hbm.at[cid].at[pl.ds(base, rows_per_tile)])

    return k


def _make_mlp(n_pad, d, bm, n_out, final):
    inv_std = float((1.0 + 1e-5) ** -0.5)

    def body(p_ref, w1_ref, b1_ref, w2_ref, b2_ref, g_ref, be_ref, o_ref):
        z = p_ref[0] + p_ref[1]
        y = jnp.dot(z, w1_ref[...], preferred_element_type=jnp.float32)
        y = jnp.maximum(y + b1_ref[...], 0.0)
        y = jnp.dot(y, w2_ref[...], preferred_element_type=jnp.float32)
        y = y + b2_ref[...]
        y = y * (g_ref[...] * inv_std) + be_ref[...]
        if not final:
            y = jnp.maximum(y, 0.0)
        o_ref[...] = y

    grid = -(-n_out // bm)
    full = lambda i: (0, 0)
    return pl.pallas_call(
        body,
        grid=(grid,),
        in_specs=[
            pl.BlockSpec((NC, bm, d), lambda i: (0, i, 0)),
            pl.BlockSpec((d, d), full),
            pl.BlockSpec((d,), lambda i: (0,)),
            pl.BlockSpec((d, d), full),
            pl.BlockSpec((d,), lambda i: (0,)),
            pl.BlockSpec((d,), lambda i: (0,)),
            pl.BlockSpec((d,), lambda i: (0,)),
        ],
        out_specs=pl.BlockSpec((bm, d), lambda i: (i, 0)),
        out_shape=jax.ShapeDtypeStruct((n_out, d), jnp.float32),
    )


def kernel(x, edge_index, params):
    n, d = x.shape
    e = edge_index.shape[1]
    n_pad = ((n + 1 + 127) // 128) * 128
    chunks = -(-e // (NW * CHUNK))
    pc = -(-chunks // PHASES)
    pc = ((pc + 1) // 2) * 2
    chunks = PHASES * pc
    e_pad = NW * chunks * CHUNK

    bm = n_pad
    while bm > 4096 or bm % 8:
        bm //= 2

    pad_idx = n + (jnp.arange(e_pad - e, dtype=jnp.int32) % (n_pad - n))
    src = jnp.concatenate([edge_index[0], pad_idx]).reshape(NW, PHASES, pc, CHUNK)
    dst = jnp.concatenate([edge_index[1], pad_idx]).reshape(NW, PHASES, pc, CHUNK)

    h = jnp.concatenate([x, jnp.zeros((n_pad - n, d), jnp.float32)])
    zeros = jnp.zeros((n_pad, d), jnp.float32)

    seg = _make_sc_segment_sum(n_pad, d, pc)
    for i, (w1, b1, w2, b2, gamma, beta) in enumerate(params):
        final = i == len(params) - 1
        p = seg(h, zeros, src, dst)
        mlp = _make_mlp(n_pad, d, bm, n if final else n_pad, final)
        h = mlp(p, w1, b1, w2, b2, gamma, beta)
    return h


# speedup vs baseline: 1.0761x; 1.0079x over previous
import functools

import jax
import jax.numpy as jnp
import numpy as np
from jax import lax
from jax.experimental import pallas as pl
from jax.experimental.pallas import tpu as pltpu
from jax.experimental.pallas import tpu_sc as plsc

NC = 2
NS = 16
NW = NC * NS
CHUNK = 128
PHASES = 2


def _make_sc_segment_sum(n_pad, d, pc):
    mesh = plsc.VectorSubcoreMesh(core_axis_name="c", subcore_axis_name="s")
    rows_per_tile = n_pad // NS

    @functools.partial(
        pl.kernel,
        out_type=jax.ShapeDtypeStruct((NC, n_pad, d), jnp.float32),
        mesh=mesh,
        scratch_types=[
            pltpu.VMEM((pc, CHUNK), jnp.int32),
            pltpu.VMEM((pc, CHUNK), jnp.int32),
            pltpu.VMEM((CHUNK, d), jnp.float32),
            pltpu.VMEM((CHUNK, d), jnp.float32),
            pltpu.VMEM_SHARED((n_pad, d), jnp.float32),
            pltpu.SemaphoreType.DMA,
            pltpu.SemaphoreType.DMA,
        ],
    )
    def k(h_hbm, zeros_hbm, edges_hbm, out_hbm,
          src_v, dst_v, buf_a, buf_b, acc, sem_a, sem_b):
        cid = lax.axis_index("c")
        sid = lax.axis_index("s")
        wid = cid * NS + sid
        base = sid * rows_per_tile
        @pl.when(cid == 0)
        def _():
            pltpu.sync_copy(h_hbm.at[pl.ds(base, rows_per_tile)],
                            acc.at[pl.ds(base, rows_per_tile)])

        @pl.when(cid != 0)
        def _():
            pltpu.sync_copy(zeros_hbm.at[pl.ds(base, rows_per_tile)],
                            acc.at[pl.ds(base, rows_per_tile)])

        plsc.subcore_barrier()

        def g_start(j, buf, sem):
            pltpu.async_copy(h_hbm.at[src_v.at[j]], buf, sem)

        def g_wait(j, buf, sem):
            pltpu.make_async_copy(h_hbm.at[src_v.at[j]], buf, sem).wait()

        def scat(j, buf):
            pltpu.sync_copy(buf, acc.at[dst_v.at[j]], add=True)

        for ph in range(PHASES):
            pltpu.sync_copy(edges_hbm.at[0].at[wid].at[ph], src_v)
            pltpu.sync_copy(edges_hbm.at[1].at[wid].at[ph], dst_v)

            g_start(0, buf_a, sem_a)

            def body(jj, carry):
                j0 = 2 * jj
                g_start(j0 + 1, buf_b, sem_b)
                g_wait(j0, buf_a, sem_a)
                scat(j0, buf_a)

                @pl.when(jj < pc // 2 - 1)
                def _():
                    g_start(j0 + 2, buf_a, sem_a)

                g_wait(j0 + 1, buf_b, sem_b)
                scat(j0 + 1, buf_b)
                return carry

            lax.fori_loop(0, pc // 2, body, 0)
        plsc.subcore_barrier()
        pltpu.sync_copy(acc.at[pl.ds(base, rows_per_tile)],
                        out_hbm.at[cid].at[pl.ds(base, rows_per_tile)])

    return k


def _make_mlp(n_pad, d, bm, n_out, final):
    inv_std = float((1.0 + 1e-5) ** -0.5)

    def body(p_ref, w1_ref, b1_ref, w2_ref, b2_ref, g_ref, be_ref, o_ref):
        z = p_ref[0] + p_ref[1]
        y = jnp.dot(z, w1_ref[...], preferred_element_type=jnp.float32)
        y = jnp.maximum(y + b1_ref[...], 0.0)
        y = jnp.dot(y, w2_ref[...], preferred_element_type=jnp.float32)
        y = y + b2_ref[...]
        y = y * (g_ref[...] * inv_std) + be_ref[...]
        if not final:
            y = jnp.maximum(y, 0.0)
        o_ref[...] = y

    grid = -(-n_out // bm)
    full = lambda i: (0, 0)
    return pl.pallas_call(
        body,
        grid=(grid,),
        in_specs=[
            pl.BlockSpec((NC, bm, d), lambda i: (0, i, 0)),
            pl.BlockSpec((d, d), full),
            pl.BlockSpec((d,), lambda i: (0,)),
            pl.BlockSpec((d, d), full),
            pl.BlockSpec((d,), lambda i: (0,)),
            pl.BlockSpec((d,), lambda i: (0,)),
            pl.BlockSpec((d,), lambda i: (0,)),
        ],
        out_specs=pl.BlockSpec((bm, d), lambda i: (i, 0)),
        out_shape=jax.ShapeDtypeStruct((n_out, d), jnp.float32),
    )


def kernel(x, edge_index, params):
    n, d = x.shape
    e = edge_index.shape[1]
    n_pad = ((n + 1 + 127) // 128) * 128
    chunks = -(-e // (NW * CHUNK))
    pc = -(-chunks // PHASES)
    pc = ((pc + 1) // 2) * 2
    chunks = PHASES * pc
    e_pad = NW * chunks * CHUNK

    bm = n_pad
    while bm > 4096 or bm % 8:
        bm //= 2

    pad_np = (n + np.arange(e_pad - e) % (n_pad - n)).astype(np.int32)
    pad_block = jnp.asarray(np.broadcast_to(pad_np, (2, e_pad - e)))
    edges = jnp.concatenate([edge_index, pad_block], axis=1)
    edges = edges.reshape(2, NW, PHASES, pc, CHUNK)

    h = jnp.concatenate([x, jnp.zeros((n_pad - n, d), jnp.float32)])
    zeros = jnp.zeros((n_pad, d), jnp.float32)

    seg = _make_sc_segment_sum(n_pad, d, pc)
    for i, (w1, b1, w2, b2, gamma, beta) in enumerate(params):
        final = i == len(params) - 1
        p = seg(h, zeros, edges)
        mlp = _make_mlp(n_pad, d, bm, n if final else n_pad, final)
        h = mlp(p, w1, b1, w2, b2, gamma, beta)
    return h
